# baseline (speedup 1.0000x reference)
import functools
import math

import jax
import jax.numpy as jnp
from jax import lax
from jax.experimental import pallas as pl
from jax.experimental.pallas import tpu as pltpu
from jax.experimental.pallas import tpu_sc as plsc

D = 128
H = 4
DH = 32
ED = 16
QD = H * ED
NC, NS = 2, 16
NW = NC * NS
CHUNK = 40
_INV_SQRT_DH = 1.0 / math.sqrt(DH)
_F32 = jnp.float32


def _tc_pre_body(dx, sx, wq, wk, wv, wet, dp_o, kv_o):
    x = dx[...]
    q = jnp.dot(x, wq[...], preferred_element_type=_F32)
    qwe = jnp.dot(q, wet[...], preferred_element_type=_F32)
    b = q.shape[0]
    dp_o[...] = jnp.concatenate([q, qwe, jnp.zeros((b, D - QD), _F32)],
                                axis=1)
    s = sx[...]
    k = jnp.dot(s, wk[...], preferred_element_type=_F32)
    v = jnp.dot(s, wv[...], preferred_element_type=_F32)
    kv_o[...] = jnp.concatenate([k, v], axis=1)


def _tc_pre(dst_x, src_x, wq, wk, wv, wet):
    n = dst_x.shape[0]
    b = 1000
    row = lambda w: pl.BlockSpec((b, w), lambda i: (i, 0))
    full = lambda s0, s1: pl.BlockSpec((s0, s1), lambda i: (0, 0))
    return pl.pallas_call(
        _tc_pre_body,
        grid=(n // b,),
        in_specs=[row(D), row(D), full(D, D), full(D, D), full(D, D),
                  full(D, QD)],
        out_specs=[row(2 * D), row(2 * D)],
        out_shape=[jax.ShapeDtypeStruct((n, 2 * D), _F32),
                   jax.ShapeDtypeStruct((n, 2 * D), _F32)],
    )(dst_x, src_x, wq, wk, wv, wet)


def _tc_post_body(out2, aacc2, wevt, w0, b0, w1, b1, w2, b2, o_ref):
    out = out2[0] + out2[1]
    af = aacc2[0] + aacc2[1]
    aacc = af[:, :QD]
    den = af[:, QD:QD + H] + 1e-9
    deninv = 1.0 / den
    b = out.shape[0]
    attn = (out.reshape(b, H, DH) * deninv[:, :, None]).reshape(b, D)
    an = (aacc.reshape(b, H, ED) * deninv[:, :, None]).reshape(b, QD)
    attn = attn + jnp.dot(an, wevt[...], preferred_element_type=_F32)
    h = jnp.dot(attn, w0[...], preferred_element_type=_F32) + b0[...]
    h = h + jax.nn.relu(jnp.dot(h, w1[...], preferred_element_type=_F32) + b1[...])
    h = h + jax.nn.relu(jnp.dot(h, w2[...], preferred_element_type=_F32) + b2[...])
    o_ref[...] = jax.nn.relu(h)


def _tc_post(out2, aacc2, wevt, w0, b0, w1, b1, w2, b2):
    n = out2.shape[1]
    b = 1000
    row3 = pl.BlockSpec((NC, b, D), lambda i: (0, i, 0))
    full = lambda s0, s1: pl.BlockSpec((s0, s1), lambda i: (0, 0))
    return pl.pallas_call(
        _tc_post_body,
        grid=(n // b,),
        in_specs=[row3, row3, full(QD, D), full(D, D),
                  full(1, D), full(D, D), full(1, D), full(D, D), full(1, D)],
        out_specs=pl.BlockSpec((b, D), lambda i: (i, 0)),
        out_shape=jax.ShapeDtypeStruct((n, D), _F32),
    )(out2, aacc2, wevt, w0, b0, w1, b1, w2, b2)


def _tc_comb_body(xa, xb, w0a, w0b, b0, w1, b1, w2, b2, o_ref):
    h = (jnp.dot(xa[...], w0a[...], preferred_element_type=_F32)
         + jnp.dot(xb[...], w0b[...], preferred_element_type=_F32) + b0[...])
    h = h + jax.nn.relu(jnp.dot(h, w1[...], preferred_element_type=_F32) + b1[...])
    h = h + jax.nn.relu(jnp.dot(h, w2[...], preferred_element_type=_F32) + b2[...])
    o_ref[...] = jax.nn.relu(h)


def _tc_comb(xa, xb, w0a, w0b, b0, w1, b1, w2, b2):
    n = xa.shape[0]
    b = 1000
    row = pl.BlockSpec((b, D), lambda i: (i, 0))
    full = lambda s0, s1: pl.BlockSpec((s0, s1), lambda i: (0, 0))
    return pl.pallas_call(
        _tc_comb_body,
        grid=(n // b,),
        in_specs=[row, row, full(D, D), full(D, D), full(1, D), full(D, D),
                  full(1, D), full(D, D), full(1, D)],
        out_specs=row,
        out_shape=jax.ShapeDtypeStruct((n, D), _F32),
    )(xa, xb, w0a, w0b, b0, w1, b1, w2, b2)


def _row_partition(n):
    main = (n // (8 * NS)) * 8
    rem = (n - main * NS) // 8
    return main, rem


_SC_PARAMS = pltpu.CompilerParams(needs_layout_passes=False)


CHUNK1 = 40


@functools.lru_cache(maxsize=None)
def _sc_pass1_factory(E, n):
    per_worker = E // NW
    n_chunks = per_worker // CHUNK1
    assert E == NW * CHUNK1 * n_chunks and n % 8 == 0 and n_chunks % 2 == 0
    main, rem = _row_partition(n)
    mesh = plsc.VectorSubcoreMesh(core_axis_name="c", subcore_axis_name="s",
                                  num_cores=NC, num_subcores=NS)
    buf2 = lambda *shape_dtype: [pltpu.VMEM(*shape_dtype) for _ in range(2)]

    @functools.partial(
        pl.kernel,
        out_type=[jax.ShapeDtypeStruct((NC, n, D), _F32),
                  jax.ShapeDtypeStruct((E * H,), _F32)],
        mesh=mesh,
        compiler_params=_SC_PARAMS,
        scratch_types=[
            buf2((CHUNK1,), jnp.int32),
            buf2((CHUNK1,), jnp.int32),
            buf2((CHUNK1, ED), _F32),
            pltpu.VMEM((CHUNK1, 2 * D), _F32),
            buf2((CHUNK1, 2 * D), _F32),
            pltpu.VMEM((CHUNK1, D), _F32),
            pltpu.VMEM((CHUNK1 * H + 16,), _F32),
            pltpu.VMEM_SHARED((n, D), _F32),
            pltpu.SemaphoreType.DMA,
            [pltpu.SemaphoreType.DMA for _ in range(2)],
        ],
    )
    def sc_pass1(dpack_hbm, kv_hbm, attr_hbm, src_hbm, dst_hbm, z_nd,
                 out_hbm, e_hbm,
                 src_idx, dst_idx, attr_v, dpack, kv, vstage, ebuf, out_sh,
                 semd, semk):
        cid = lax.axis_index("c")
        sid = lax.axis_index("s")
        wid = sid * NC + cid
        r0 = pl.multiple_of(sid * main, 8)
        rt = pl.multiple_of(main * NS + sid * 8, 8)

        def _row_sliced(fn):
            fn(r0, main)
            if rem:
                @pl.when(sid < rem)
                def _():
                    fn(rt, 8)

        _row_sliced(lambda o, s: pltpu.sync_copy(
            z_nd.at[pl.ds(o, s)], out_sh.at[pl.ds(o, s)]))
        plsc.subcore_barrier()

        lanes = lax.iota(jnp.int32, 16)
        onehot = [jnp.where(lanes == i, jnp.float32(1.0), jnp.float32(0.0))
                  for i in range(16)]

        def _prefetch(c, b):
            base = wid * per_worker + c * CHUNK1
            pltpu.sync_copy(src_hbm.at[pl.ds(base, CHUNK1)], src_idx[b])
            pltpu.sync_copy(dst_hbm.at[pl.ds(base, CHUNK1)], dst_idx[b])
            pltpu.sync_copy(attr_hbm.at[pl.ds(base, CHUNK1)], attr_v[b])
            pltpu.async_copy(kv_hbm.at[src_idx[b]], kv[b], semk[b])

        _PIPE = True
        if _PIPE:
            for b in range(2):
                _prefetch(b, b)
            pltpu.async_copy(dpack_hbm.at[dst_idx[0]], dpack, semd)

        @pl.loop(0, n_chunks // 2)
        def _chunk(j):
            for b in range(2):
                c = 2 * j + b
                base = wid * per_worker + c * CHUNK1
                if _PIPE:
                    pltpu.make_async_copy(dpack_hbm.at[dst_idx[b]], dpack,
                                          semd).wait()
                    pltpu.make_async_copy(kv_hbm.at[src_idx[b]], kv[b],
                                          semk[b]).wait()
                else:
                    _prefetch(c, b)
                    pltpu.async_copy(dpack_hbm.at[dst_idx[b]], dpack,
                                     semd).wait()
                    pltpu.make_async_copy(kv_hbm.at[src_idx[b]], kv[b],
                                          semk[b]).wait()

                @pl.loop(0, (CHUNK1 + 15) // 16)
                def _logit(gg):
                    rows0 = 16 * gg + lanes
                    m = rows0 < CHUNK1
                    rows = jnp.minimum(rows0, CHUNK1 - 1)
                    for h in range(H):
                        parts = [jnp.zeros((16,), _F32) for _ in range(4)]
                        for d in range(DH):
                            col = jnp.full((16,), DH * h + d, jnp.int32)
                            q = plsc.load_gather(dpack, [rows, col], mask=m)
                            k = plsc.load_gather(kv[b], [rows, col], mask=m)
                            parts[d % 4] = parts[d % 4] + q * k
                        for j in range(ED):
                            cq = jnp.full((16,), D + ED * h + j, jnp.int32)
                            ca = jnp.full((16,), j, jnp.int32)
                            qw = plsc.load_gather(dpack, [rows, cq], mask=m)
                            at = plsc.load_gather(attr_v[b], [rows, ca],
                                                  mask=m)
                            parts[j % 4] = parts[j % 4] + qw * at
                        acc = (parts[0] + parts[1]) + (parts[2] + parts[3])
                        e_h = jnp.exp(acc * _INV_SQRT_DH)
                        plsc.store_scatter(ebuf, [rows * H + h], e_h,
                                           mask=m)

                if _PIPE:
                    @pl.when(c + 1 < n_chunks)
                    def _():
                        pltpu.async_copy(dpack_hbm.at[dst_idx[1 - b]], dpack,
                                         semd)

                pltpu.sync_copy(ebuf.at[pl.ds(0, CHUNK1 * H)],
                                e_hbm.at[pl.ds(base * H, CHUNK1 * H)])

                @pl.loop(0, CHUNK1)
                def _scale(e):
                    ev = ebuf[pl.ds(H * e, 16)]
                    for h in range(H):
                        s = ev[h]
                        vstage[e, pl.ds(DH * h, 16)] = (
                            kv[b][e, pl.ds(D + DH * h, 16)] * s)
                        vstage[e, pl.ds(DH * h + 16, 16)] = (
                            kv[b][e, pl.ds(D + DH * h + 16, 16)] * s)

                pltpu.sync_copy(vstage, out_sh.at[dst_idx[b]], add=True)

                if _PIPE:
                    @pl.when(c + 2 < n_chunks)
                    def _():
                        _prefetch(c + 2, b)

        plsc.subcore_barrier()
        _row_sliced(lambda o, s: pltpu.sync_copy(
            out_sh.at[pl.ds(o, s)], out_hbm.at[cid, pl.ds(o, s)]))

    return sc_pass1


@functools.lru_cache(maxsize=None)
def _sc_pass2_factory(E, n):
    per_worker = E // NW
    n_chunks = per_worker // CHUNK
    assert E == NW * CHUNK * n_chunks and n % 8 == 0 and n_chunks % 2 == 0
    main, rem = _row_partition(n)
    mesh = plsc.VectorSubcoreMesh(core_axis_name="c", subcore_axis_name="s",
                                  num_cores=NC, num_subcores=NS)

    buf2 = lambda *shape_dtype: [pltpu.VMEM(*shape_dtype) for _ in range(2)]

    @functools.partial(
        pl.kernel,
        out_type=jax.ShapeDtypeStruct((NC, n, D), _F32),
        mesh=mesh,
        compiler_params=_SC_PARAMS,
        scratch_types=[
            buf2((CHUNK,), jnp.int32),
            buf2((CHUNK, ED), _F32),
            buf2((CHUNK * H + 16,), _F32),
            pltpu.VMEM((CHUNK, D), _F32),
            pltpu.VMEM_SHARED((n, D), _F32),
            [pltpu.SemaphoreType.DMA for _ in range(2)],
        ],
    )
    def sc_pass2(attr_hbm, dst_hbm, e_hbm, z_nd,
                 aacc_hbm,
                 dst_idx, attr_v, ebuf, stage, acc_sh, sem):
        cid = lax.axis_index("c")
        sid = lax.axis_index("s")
        wid = sid * NC + cid
        r0 = pl.multiple_of(sid * main, 8)
        rt = pl.multiple_of(main * NS + sid * 8, 8)

        def _row_sliced(fn):
            fn(r0, main)
            if rem:
                @pl.when(sid < rem)
                def _():
                    fn(rt, 8)

        _row_sliced(lambda o, s: pltpu.sync_copy(
            z_nd.at[pl.ds(o, s)], acc_sh.at[pl.ds(o, s)]))

        lanes = lax.iota(jnp.int32, 16)
        m4 = lanes < H
        zv = jnp.zeros((16,), _F32)

        @pl.loop(0, CHUNK)
        def _z(e):
            for c in range(QD + 16, D, 16):
                stage[e, pl.ds(c, 16)] = zv

        plsc.subcore_barrier()

        def _copies(c, b):
            base = wid * per_worker + c * CHUNK
            return [(dst_hbm.at[pl.ds(base, CHUNK)], dst_idx[b]),
                    (attr_hbm.at[pl.ds(base, CHUNK)], attr_v[b]),
                    (e_hbm.at[pl.ds(base * H, CHUNK * H)],
                     ebuf[b].at[pl.ds(0, CHUNK * H)])]

        for b in range(2):
            for s_ref, d_ref in _copies(b, b):
                pltpu.async_copy(s_ref, d_ref, sem[b])

        @pl.loop(0, n_chunks // 2)
        def _chunk(j):
            for b in range(2):
                c = 2 * j + b
                for s_ref, d_ref in _copies(c, b):
                    pltpu.make_async_copy(s_ref, d_ref, sem[b]).wait()

                @pl.loop(0, CHUNK)
                def _stage(e):
                    ev = ebuf[b][pl.ds(H * e, 16)]
                    a = attr_v[b][e, :]
                    for h in range(H):
                        stage[e, pl.ds(ED * h, 16)] = a * ev[h]
                    stage[e, pl.ds(QD, 16)] = jnp.where(m4, ev, 0.0)

                pltpu.sync_copy(stage, acc_sh.at[dst_idx[b]], add=True)

                @pl.when(c + 2 < n_chunks)
                def _():
                    for s_ref, d_ref in _copies(c + 2, b):
                        pltpu.async_copy(s_ref, d_ref, sem[b])

        plsc.subcore_barrier()
        _row_sliced(lambda o, s: pltpu.sync_copy(
            acc_sh.at[pl.ds(o, s)], aacc_hbm.at[cid, pl.ds(o, s)]))

    return sc_pass2


def _blockdiag(we, wev):
    wet = jnp.zeros((D, QD), _F32)
    wevt = jnp.zeros((QD, D), _F32)
    for h in range(H):
        wet = wet.at[h * DH:(h + 1) * DH, h * ED:(h + 1) * ED].set(
            we[:, h * DH:(h + 1) * DH].T)
        wevt = wevt.at[h * ED:(h + 1) * ED, h * DH:(h + 1) * DH].set(
            wev[:, h * DH:(h + 1) * DH])
    return wet, wevt


def _att_layer(sc1, sc2, src_x, dst_x, attr, eidx, p, z_nd):
    wet, wevt = _blockdiag(p['We'], p['Wev'])
    dpack, kvpack = _tc_pre(dst_x, src_x, p['Wq'], p['Wk'], p['Wv'], wet)
    out2, e_w = sc1(dpack, kvpack, attr, eidx[0], eidx[1], z_nd)
    aacc2 = sc2(attr, eidx[1], e_w, z_nd)
    r = p['res']
    return _tc_post(out2, aacc2, wevt,
                    r['W0'], r['b0'].reshape(1, D),
                    r['Ws'][0], r['bs'][0].reshape(1, D),
                    r['Ws'][1], r['bs'][1].reshape(1, D))


def kernel(lane_x, movement_x, lane_to_downstream_movement_edge_attr,
           lane_to_upstream_movement_edge_attr, movement_to_movement_edge_attr,
           lane_to_downstream_movement_edge_index,
           lane_to_upstream_movement_edge_index,
           movement_to_movement_edge_index, params):
    n = movement_x.shape[0]
    e = lane_to_downstream_movement_edge_attr.shape[0]
    sc1 = _sc_pass1_factory(e, n)
    sc2 = _sc_pass2_factory(e, n)
    z_nd = jnp.zeros((n, D), _F32)
    inc = _att_layer(sc1, sc2, lane_x, movement_x,
                     lane_to_downstream_movement_edge_attr,
                     lane_to_downstream_movement_edge_index, params['inc'],
                     z_nd)
    outg = _att_layer(sc1, sc2, lane_x, movement_x,
                      lane_to_upstream_movement_edge_attr,
                      lane_to_upstream_movement_edge_index, params['outg'],
                      z_nd)
    c = params['comb']
    mov = _tc_comb(inc, outg, c['W0'][:D], c['W0'][D:],
                   c['b0'].reshape(1, D), c['Ws'][0], c['bs'][0].reshape(1, D),
                   c['Ws'][1], c['bs'][1].reshape(1, D))
    for hp in params['hops']:
        mov = _att_layer(sc1, sc2, mov, mov, movement_to_movement_edge_attr,
                         movement_to_movement_edge_index, hp, z_nd)
    return mov


# speedup vs baseline: 2.1549x; 2.1549x over previous
import functools
import math

import jax
import jax.numpy as jnp
from jax import lax
from jax.experimental import pallas as pl
from jax.experimental.pallas import tpu as pltpu
from jax.experimental.pallas import tpu_sc as plsc

D = 128
H = 4
DH = 32
ED = 16
QD = H * ED
NC, NS = 2, 16
NW = NC * NS
CHUNK = 40
_INV_SQRT_DH = 1.0 / math.sqrt(DH)
_F32 = jnp.float32


def _tc_pre_body(dx, sx, wq, wk, wv, wet, dp_o, kv_o):
    x = dx[...]
    q = jnp.dot(x, wq[...], preferred_element_type=_F32)
    qwe = jnp.dot(q, wet[...], preferred_element_type=_F32)
    b = q.shape[0]
    dp_o[...] = jnp.concatenate([q, qwe, jnp.zeros((b, D - QD), _F32)],
                                axis=1)
    s = sx[...]
    k = jnp.dot(s, wk[...], preferred_element_type=_F32)
    v = jnp.dot(s, wv[...], preferred_element_type=_F32)
    kv_o[...] = jnp.concatenate([k, v], axis=1)


def _tc_pre(dst_x, src_x, wq, wk, wv, wet):
    n = dst_x.shape[0]
    b = 1000
    row = lambda w: pl.BlockSpec((b, w), lambda i: (i, 0))
    full = lambda s0, s1: pl.BlockSpec((s0, s1), lambda i: (0, 0))
    return pl.pallas_call(
        _tc_pre_body,
        grid=(n // b,),
        in_specs=[row(D), row(D), full(D, D), full(D, D), full(D, D),
                  full(D, QD)],
        out_specs=[row(2 * D), row(2 * D)],
        out_shape=[jax.ShapeDtypeStruct((n, 2 * D), _F32),
                   jax.ShapeDtypeStruct((n, 2 * D), _F32)],
    )(dst_x, src_x, wq, wk, wv, wet)


def _tc_post_body(out2, aacc2, wevt, w0, b0, w1, b1, w2, b2, o_ref):
    out = out2[0] + out2[1]
    af = aacc2[0] + aacc2[1]
    aacc = af[:, :QD]
    den = af[:, QD:QD + H] + 1e-9
    deninv = 1.0 / den
    b = out.shape[0]
    attn = (out.reshape(b, H, DH) * deninv[:, :, None]).reshape(b, D)
    an = (aacc.reshape(b, H, ED) * deninv[:, :, None]).reshape(b, QD)
    attn = attn + jnp.dot(an, wevt[...], preferred_element_type=_F32)
    h = jnp.dot(attn, w0[...], preferred_element_type=_F32) + b0[...]
    h = h + jax.nn.relu(jnp.dot(h, w1[...], preferred_element_type=_F32) + b1[...])
    h = h + jax.nn.relu(jnp.dot(h, w2[...], preferred_element_type=_F32) + b2[...])
    o_ref[...] = jax.nn.relu(h)


def _tc_post(out2, aacc2, wevt, w0, b0, w1, b1, w2, b2):
    n = out2.shape[1]
    b = 1000
    row3 = pl.BlockSpec((NC, b, D), lambda i: (0, i, 0))
    full = lambda s0, s1: pl.BlockSpec((s0, s1), lambda i: (0, 0))
    return pl.pallas_call(
        _tc_post_body,
        grid=(n // b,),
        in_specs=[row3, row3, full(QD, D), full(D, D),
                  full(1, D), full(D, D), full(1, D), full(D, D), full(1, D)],
        out_specs=pl.BlockSpec((b, D), lambda i: (i, 0)),
        out_shape=jax.ShapeDtypeStruct((n, D), _F32),
    )(out2, aacc2, wevt, w0, b0, w1, b1, w2, b2)


def _tc_comb_body(xa, xb, w0a, w0b, b0, w1, b1, w2, b2, o_ref):
    h = (jnp.dot(xa[...], w0a[...], preferred_element_type=_F32)
         + jnp.dot(xb[...], w0b[...], preferred_element_type=_F32) + b0[...])
    h = h + jax.nn.relu(jnp.dot(h, w1[...], preferred_element_type=_F32) + b1[...])
    h = h + jax.nn.relu(jnp.dot(h, w2[...], preferred_element_type=_F32) + b2[...])
    o_ref[...] = jax.nn.relu(h)


def _tc_comb(xa, xb, w0a, w0b, b0, w1, b1, w2, b2):
    n = xa.shape[0]
    b = 1000
    row = pl.BlockSpec((b, D), lambda i: (i, 0))
    full = lambda s0, s1: pl.BlockSpec((s0, s1), lambda i: (0, 0))
    return pl.pallas_call(
        _tc_comb_body,
        grid=(n // b,),
        in_specs=[row, row, full(D, D), full(D, D), full(1, D), full(D, D),
                  full(1, D), full(D, D), full(1, D)],
        out_specs=row,
        out_shape=jax.ShapeDtypeStruct((n, D), _F32),
    )(xa, xb, w0a, w0b, b0, w1, b1, w2, b2)


def _row_partition(n):
    main = (n // (8 * NS)) * 8
    rem = (n - main * NS) // 8
    return main, rem


_SC_PARAMS = pltpu.CompilerParams(needs_layout_passes=False)


CHUNK1 = 40


@functools.lru_cache(maxsize=None)
def _sc_pass1_factory(E, n):
    per_worker = E // NW
    n_chunks = per_worker // CHUNK1
    assert E == NW * CHUNK1 * n_chunks and n % 8 == 0 and n_chunks % 2 == 0
    main, rem = _row_partition(n)
    mesh = plsc.VectorSubcoreMesh(core_axis_name="c", subcore_axis_name="s",
                                  num_cores=NC, num_subcores=NS)
    buf2 = lambda *shape_dtype: [pltpu.VMEM(*shape_dtype) for _ in range(2)]

    @functools.partial(
        pl.kernel,
        out_type=[jax.ShapeDtypeStruct((NC, n, D), _F32),
                  jax.ShapeDtypeStruct((E * H,), _F32)],
        mesh=mesh,
        compiler_params=_SC_PARAMS,
        scratch_types=[
            buf2((CHUNK1,), jnp.int32),
            buf2((CHUNK1,), jnp.int32),
            buf2((CHUNK1, ED), _F32),
            pltpu.VMEM((CHUNK1, 2 * D), _F32),
            buf2((CHUNK1, 2 * D), _F32),
            pltpu.VMEM((CHUNK1, D), _F32),
            pltpu.VMEM((CHUNK1 * H + 16,), _F32),
            pltpu.VMEM_SHARED((n, D), _F32),
            pltpu.SemaphoreType.DMA,
            [pltpu.SemaphoreType.DMA for _ in range(2)],
        ],
    )
    def sc_pass1(dpack_hbm, kv_hbm, attr_hbm, src_hbm, dst_hbm, z_nd,
                 out_hbm, e_hbm,
                 src_idx, dst_idx, attr_v, dpack, kv, vstage, ebuf, out_sh,
                 semd, semk):
        cid = lax.axis_index("c")
        sid = lax.axis_index("s")
        wid = sid * NC + cid
        r0 = pl.multiple_of(sid * main, 8)
        rt = pl.multiple_of(main * NS + sid * 8, 8)

        def _row_sliced(fn):
            fn(r0, main)
            if rem:
                @pl.when(sid < rem)
                def _():
                    fn(rt, 8)

        _row_sliced(lambda o, s: pltpu.sync_copy(
            z_nd.at[pl.ds(o, s)], out_sh.at[pl.ds(o, s)]))
        plsc.subcore_barrier()

        lanes = lax.iota(jnp.int32, 16)

        def _prefetch(c, b):
            base = wid * per_worker + c * CHUNK1
            pltpu.sync_copy(src_hbm.at[pl.ds(base, CHUNK1)], src_idx[b])
            pltpu.sync_copy(dst_hbm.at[pl.ds(base, CHUNK1)], dst_idx[b])
            pltpu.sync_copy(attr_hbm.at[pl.ds(base, CHUNK1)], attr_v[b])
            pltpu.async_copy(kv_hbm.at[src_idx[b]], kv[b], semk[b])

        for b in range(2):
            _prefetch(b, b)
        pltpu.async_copy(dpack_hbm.at[dst_idx[0]], dpack, semd)

        @pl.loop(0, n_chunks // 2)
        def _chunk(j):
            for b in range(2):
                c = 2 * j + b
                base = wid * per_worker + c * CHUNK1
                pltpu.make_async_copy(dpack_hbm.at[dst_idx[b]], dpack,
                                      semd).wait()
                pltpu.make_async_copy(kv_hbm.at[src_idx[b]], kv[b],
                                      semk[b]).wait()

                @pl.loop(0, CHUNK1 // 4)
                def _logit(g):
                    vals = jnp.zeros((16,), _F32)
                    for q4 in range(4):
                        e = g * 4 + q4
                        a = attr_v[b][e, :]
                        for h in range(H):
                            p = (dpack[e, pl.ds(DH * h, 16)]
                                 * kv[b][e, pl.ds(DH * h, 16)]
                                 + dpack[e, pl.ds(DH * h + 16, 16)]
                                 * kv[b][e, pl.ds(DH * h + 16, 16)]
                                 + dpack[e, pl.ds(D + ED * h, 16)] * a)
                            vals = jnp.where(lanes == (q4 * H + h),
                                             jnp.sum(p), vals)
                    ebuf[pl.ds(16 * g, 16)] = vals

                @pl.when(c + 1 < n_chunks)
                def _():
                    pltpu.async_copy(dpack_hbm.at[dst_idx[1 - b]], dpack,
                                     semd)

                @pl.loop(0, CHUNK1 * H // 16)
                def _exp(jj):
                    x = ebuf[pl.ds(jj * 16, 16)]
                    ebuf[pl.ds(jj * 16, 16)] = jnp.exp(x * _INV_SQRT_DH)

                pltpu.sync_copy(ebuf.at[pl.ds(0, CHUNK1 * H)],
                                e_hbm.at[pl.ds(base * H, CHUNK1 * H)])

                @pl.loop(0, CHUNK1)
                def _scale(e):
                    ev = ebuf[pl.ds(H * e, 16)]
                    for h in range(H):
                        s = ev[h]
                        vstage[e, pl.ds(DH * h, 16)] = (
                            kv[b][e, pl.ds(D + DH * h, 16)] * s)
                        vstage[e, pl.ds(DH * h + 16, 16)] = (
                            kv[b][e, pl.ds(D + DH * h + 16, 16)] * s)

                pltpu.sync_copy(vstage, out_sh.at[dst_idx[b]], add=True)

                @pl.when(c + 2 < n_chunks)
                def _():
                    _prefetch(c + 2, b)

        plsc.subcore_barrier()
        _row_sliced(lambda o, s: pltpu.sync_copy(
            out_sh.at[pl.ds(o, s)], out_hbm.at[cid, pl.ds(o, s)]))

    return sc_pass1


@functools.lru_cache(maxsize=None)
def _sc_pass2_factory(E, n):
    per_worker = E // NW
    n_chunks = per_worker // CHUNK
    assert E == NW * CHUNK * n_chunks and n % 8 == 0 and n_chunks % 2 == 0
    main, rem = _row_partition(n)
    mesh = plsc.VectorSubcoreMesh(core_axis_name="c", subcore_axis_name="s",
                                  num_cores=NC, num_subcores=NS)

    buf2 = lambda *shape_dtype: [pltpu.VMEM(*shape_dtype) for _ in range(2)]

    @functools.partial(
        pl.kernel,
        out_type=jax.ShapeDtypeStruct((NC, n, D), _F32),
        mesh=mesh,
        compiler_params=_SC_PARAMS,
        scratch_types=[
            buf2((CHUNK,), jnp.int32),
            buf2((CHUNK, ED), _F32),
            buf2((CHUNK * H + 16,), _F32),
            pltpu.VMEM((CHUNK, D), _F32),
            pltpu.VMEM_SHARED((n, D), _F32),
            [pltpu.SemaphoreType.DMA for _ in range(2)],
        ],
    )
    def sc_pass2(attr_hbm, dst_hbm, e_hbm, z_nd,
                 aacc_hbm,
                 dst_idx, attr_v, ebuf, stage, acc_sh, sem):
        cid = lax.axis_index("c")
        sid = lax.axis_index("s")
        wid = sid * NC + cid
        r0 = pl.multiple_of(sid * main, 8)
        rt = pl.multiple_of(main * NS + sid * 8, 8)

        def _row_sliced(fn):
            fn(r0, main)
            if rem:
                @pl.when(sid < rem)
                def _():
                    fn(rt, 8)

        _row_sliced(lambda o, s: pltpu.sync_copy(
            z_nd.at[pl.ds(o, s)], acc_sh.at[pl.ds(o, s)]))

        lanes = lax.iota(jnp.int32, 16)
        m4 = lanes < H
        zv = jnp.zeros((16,), _F32)

        @pl.loop(0, CHUNK)
        def _z(e):
            for c in range(QD + 16, D, 16):
                stage[e, pl.ds(c, 16)] = zv

        plsc.subcore_barrier()

        def _copies(c, b):
            base = wid * per_worker + c * CHUNK
            return [(dst_hbm.at[pl.ds(base, CHUNK)], dst_idx[b]),
                    (attr_hbm.at[pl.ds(base, CHUNK)], attr_v[b]),
                    (e_hbm.at[pl.ds(base * H, CHUNK * H)],
                     ebuf[b].at[pl.ds(0, CHUNK * H)])]

        for b in range(2):
            for s_ref, d_ref in _copies(b, b):
                pltpu.async_copy(s_ref, d_ref, sem[b])

        @pl.loop(0, n_chunks // 2)
        def _chunk(j):
            for b in range(2):
                c = 2 * j + b
                for s_ref, d_ref in _copies(c, b):
                    pltpu.make_async_copy(s_ref, d_ref, sem[b]).wait()

                @pl.loop(0, CHUNK)
                def _stage(e):
                    ev = ebuf[b][pl.ds(H * e, 16)]
                    a = attr_v[b][e, :]
                    for h in range(H):
                        stage[e, pl.ds(ED * h, 16)] = a * ev[h]
                    stage[e, pl.ds(QD, 16)] = jnp.where(m4, ev, 0.0)

                pltpu.sync_copy(stage, acc_sh.at[dst_idx[b]], add=True)

                @pl.when(c + 2 < n_chunks)
                def _():
                    for s_ref, d_ref in _copies(c + 2, b):
                        pltpu.async_copy(s_ref, d_ref, sem[b])

        plsc.subcore_barrier()
        _row_sliced(lambda o, s: pltpu.sync_copy(
            acc_sh.at[pl.ds(o, s)], aacc_hbm.at[cid, pl.ds(o, s)]))

    return sc_pass2


def _blockdiag(we, wev):
    wet = jnp.zeros((D, QD), _F32)
    wevt = jnp.zeros((QD, D), _F32)
    for h in range(H):
        wet = wet.at[h * DH:(h + 1) * DH, h * ED:(h + 1) * ED].set(
            we[:, h * DH:(h + 1) * DH].T)
        wevt = wevt.at[h * ED:(h + 1) * ED, h * DH:(h + 1) * DH].set(
            wev[:, h * DH:(h + 1) * DH])
    return wet, wevt


def _att_layer(sc1, sc2, src_x, dst_x, attr, eidx, p, z_nd):
    wet, wevt = _blockdiag(p['We'], p['Wev'])
    dpack, kvpack = _tc_pre(dst_x, src_x, p['Wq'], p['Wk'], p['Wv'], wet)
    out2, e_w = sc1(dpack, kvpack, attr, eidx[0], eidx[1], z_nd)
    aacc2 = sc2(attr, eidx[1], e_w, z_nd)
    r = p['res']
    return _tc_post(out2, aacc2, wevt,
                    r['W0'], r['b0'].reshape(1, D),
                    r['Ws'][0], r['bs'][0].reshape(1, D),
                    r['Ws'][1], r['bs'][1].reshape(1, D))


def kernel(lane_x, movement_x, lane_to_downstream_movement_edge_attr,
           lane_to_upstream_movement_edge_attr, movement_to_movement_edge_attr,
           lane_to_downstream_movement_edge_index,
           lane_to_upstream_movement_edge_index,
           movement_to_movement_edge_index, params):
    n = movement_x.shape[0]
    e = lane_to_downstream_movement_edge_attr.shape[0]
    sc1 = _sc_pass1_factory(e, n)
    sc2 = _sc_pass2_factory(e, n)
    z_nd = jnp.zeros((n, D), _F32)
    inc = _att_layer(sc1, sc2, lane_x, movement_x,
                     lane_to_downstream_movement_edge_attr,
                     lane_to_downstream_movement_edge_index, params['inc'],
                     z_nd)
    outg = _att_layer(sc1, sc2, lane_x, movement_x,
                      lane_to_upstream_movement_edge_attr,
                      lane_to_upstream_movement_edge_index, params['outg'],
                      z_nd)
    c = params['comb']
    mov = _tc_comb(inc, outg, c['W0'][:D], c['W0'][D:],
                   c['b0'].reshape(1, D), c['Ws'][0], c['bs'][0].reshape(1, D),
                   c['Ws'][1], c['bs'][1].reshape(1, D))
    for hp in params['hops']:
        mov = _att_layer(sc1, sc2, mov, mov, movement_to_movement_edge_attr,
                         movement_to_movement_edge_index, hp, z_nd)
    return mov


# speedup vs baseline: 2.5821x; 1.1982x over previous
import functools
import math

import jax
import jax.numpy as jnp
from jax import lax
from jax.experimental import pallas as pl
from jax.experimental.pallas import tpu as pltpu
from jax.experimental.pallas import tpu_sc as plsc

D = 128
H = 4
DH = 32
ED = 16
QD = H * ED
NC, NS = 2, 16
NW = NC * NS
CHUNK = 40
_INV_SQRT_DH = 1.0 / math.sqrt(DH)
_F32 = jnp.float32


def _tc_pre_body(dx, sx, wq, wk, wv, wet, dp_o, kv_o):
    x = dx[...]
    q = jnp.dot(x, wq[...], preferred_element_type=_F32)
    qwe = jnp.dot(q, wet[...], preferred_element_type=_F32)
    b = q.shape[0]
    dp_o[...] = jnp.concatenate([q, qwe, jnp.zeros((b, D - QD), _F32)],
                                axis=1)
    s = sx[...]
    k = jnp.dot(s, wk[...], preferred_element_type=_F32)
    v = jnp.dot(s, wv[...], preferred_element_type=_F32)
    kv_o[...] = jnp.concatenate([k, v], axis=1)


def _tc_pre(dst_x, src_x, wq, wk, wv, wet):
    n = dst_x.shape[0]
    b = 1000
    row = lambda w: pl.BlockSpec((b, w), lambda i: (i, 0))
    full = lambda s0, s1: pl.BlockSpec((s0, s1), lambda i: (0, 0))
    return pl.pallas_call(
        _tc_pre_body,
        grid=(n // b,),
        in_specs=[row(D), row(D), full(D, D), full(D, D), full(D, D),
                  full(D, QD)],
        out_specs=[row(2 * D), row(2 * D)],
        out_shape=[jax.ShapeDtypeStruct((n, 2 * D), _F32),
                   jax.ShapeDtypeStruct((n, 2 * D), _F32)],
    )(dst_x, src_x, wq, wk, wv, wet)


def _tc_post_body(out2, aacc2, wevt, w0, b0, w1, b1, w2, b2, o_ref):
    out = out2[0] + out2[1]
    af = aacc2[0] + aacc2[1]
    aacc = af[:, :QD]
    den = af[:, QD:QD + H] + 1e-9
    deninv = 1.0 / den
    b = out.shape[0]
    attn = (out.reshape(b, H, DH) * deninv[:, :, None]).reshape(b, D)
    an = (aacc.reshape(b, H, ED) * deninv[:, :, None]).reshape(b, QD)
    attn = attn + jnp.dot(an, wevt[...], preferred_element_type=_F32)
    h = jnp.dot(attn, w0[...], preferred_element_type=_F32) + b0[...]
    h = h + jax.nn.relu(jnp.dot(h, w1[...], preferred_element_type=_F32) + b1[...])
    h = h + jax.nn.relu(jnp.dot(h, w2[...], preferred_element_type=_F32) + b2[...])
    o_ref[...] = jax.nn.relu(h)


def _tc_post(out2, aacc2, wevt, w0, b0, w1, b1, w2, b2):
    n = out2.shape[1]
    b = 1000
    row3 = pl.BlockSpec((NC, b, D), lambda i: (0, i, 0))
    full = lambda s0, s1: pl.BlockSpec((s0, s1), lambda i: (0, 0))
    return pl.pallas_call(
        _tc_post_body,
        grid=(n // b,),
        in_specs=[row3, row3, full(QD, D), full(D, D),
                  full(1, D), full(D, D), full(1, D), full(D, D), full(1, D)],
        out_specs=pl.BlockSpec((b, D), lambda i: (i, 0)),
        out_shape=jax.ShapeDtypeStruct((n, D), _F32),
    )(out2, aacc2, wevt, w0, b0, w1, b1, w2, b2)


def _tc_comb_body(xa, xb, w0a, w0b, b0, w1, b1, w2, b2, o_ref):
    h = (jnp.dot(xa[...], w0a[...], preferred_element_type=_F32)
         + jnp.dot(xb[...], w0b[...], preferred_element_type=_F32) + b0[...])
    h = h + jax.nn.relu(jnp.dot(h, w1[...], preferred_element_type=_F32) + b1[...])
    h = h + jax.nn.relu(jnp.dot(h, w2[...], preferred_element_type=_F32) + b2[...])
    o_ref[...] = jax.nn.relu(h)


def _tc_comb(xa, xb, w0a, w0b, b0, w1, b1, w2, b2):
    n = xa.shape[0]
    b = 1000
    row = pl.BlockSpec((b, D), lambda i: (i, 0))
    full = lambda s0, s1: pl.BlockSpec((s0, s1), lambda i: (0, 0))
    return pl.pallas_call(
        _tc_comb_body,
        grid=(n // b,),
        in_specs=[row, row, full(D, D), full(D, D), full(1, D), full(D, D),
                  full(1, D), full(D, D), full(1, D)],
        out_specs=row,
        out_shape=jax.ShapeDtypeStruct((n, D), _F32),
    )(xa, xb, w0a, w0b, b0, w1, b1, w2, b2)


def _row_partition(n):
    main = (n // (8 * NS)) * 8
    rem = (n - main * NS) // 8
    return main, rem


_SC_PARAMS = pltpu.CompilerParams(needs_layout_passes=False)


CHUNK1 = 40


@functools.lru_cache(maxsize=None)
def _sc_pass1_factory(E, n):
    per_worker = E // NW
    n_chunks = per_worker // CHUNK1
    assert E == NW * CHUNK1 * n_chunks and n % 8 == 0 and n_chunks % 2 == 0
    main, rem = _row_partition(n)
    mesh = plsc.VectorSubcoreMesh(core_axis_name="c", subcore_axis_name="s",
                                  num_cores=NC, num_subcores=NS)
    buf2 = lambda *shape_dtype: [pltpu.VMEM(*shape_dtype) for _ in range(2)]

    @functools.partial(
        pl.kernel,
        out_type=[jax.ShapeDtypeStruct((NC, n, D), _F32),
                  jax.ShapeDtypeStruct((E * H,), _F32)],
        mesh=mesh,
        compiler_params=_SC_PARAMS,
        scratch_types=[
            buf2((CHUNK1,), jnp.int32),
            buf2((CHUNK1,), jnp.int32),
            buf2((CHUNK1, ED), _F32),
            pltpu.VMEM((CHUNK1, 2 * D), _F32),
            buf2((CHUNK1, 2 * D), _F32),
            pltpu.VMEM((CHUNK1, D), _F32),
            pltpu.VMEM((CHUNK1 * H + 16,), _F32),
            pltpu.VMEM_SHARED((n, D), _F32),
            pltpu.SemaphoreType.DMA,
            [pltpu.SemaphoreType.DMA for _ in range(2)],
            pltpu.SemaphoreType.DMA,
            pltpu.SemaphoreType.DMA,
        ],
    )
    def sc_pass1(dpack_hbm, kv_hbm, attr_hbm, src_hbm, dst_hbm, z_nd,
                 out_hbm, e_hbm,
                 src_idx, dst_idx, attr_v, dpack, kv, vstage, ebuf, out_sh,
                 semd, semk, semp, seme):
        cid = lax.axis_index("c")
        sid = lax.axis_index("s")
        wid = sid * NC + cid
        r0 = pl.multiple_of(sid * main, 8)
        rt = pl.multiple_of(main * NS + sid * 8, 8)

        def _row_sliced(fn):
            fn(r0, main)
            if rem:
                @pl.when(sid < rem)
                def _():
                    fn(rt, 8)

        _row_sliced(lambda o, s: pltpu.sync_copy(
            z_nd.at[pl.ds(o, s)], out_sh.at[pl.ds(o, s)]))
        plsc.subcore_barrier()

        lanes = lax.iota(jnp.int32, 16)

        def _prefetch(c, b):
            base = wid * per_worker + c * CHUNK1
            c1 = pltpu.async_copy(src_hbm.at[pl.ds(base, CHUNK1)],
                                  src_idx[b], semp)
            c2 = pltpu.async_copy(dst_hbm.at[pl.ds(base, CHUNK1)],
                                  dst_idx[b], semp)
            c3 = pltpu.async_copy(attr_hbm.at[pl.ds(base, CHUNK1)],
                                  attr_v[b], semp)
            c1.wait()
            c2.wait()
            c3.wait()
            pltpu.async_copy(kv_hbm.at[src_idx[b]], kv[b], semk[b])

        for b in range(2):
            _prefetch(b, b)
        pltpu.async_copy(dpack_hbm.at[dst_idx[0]], dpack, semd)

        @pl.loop(0, n_chunks // 2)
        def _chunk(j):
            for b in range(2):
                c = 2 * j + b
                base = wid * per_worker + c * CHUNK1
                pltpu.make_async_copy(dpack_hbm.at[dst_idx[b]], dpack,
                                      semd).wait()
                pltpu.make_async_copy(kv_hbm.at[src_idx[b]], kv[b],
                                      semk[b]).wait()

                @pl.when(c > 0)
                def _():
                    pltpu.make_async_copy(
                        ebuf.at[pl.ds(0, CHUNK1 * H)],
                        e_hbm.at[pl.ds(base * H, CHUNK1 * H)], seme).wait()

                @pl.loop(0, CHUNK1 // 4)
                def _logit(g):
                    vals = jnp.zeros((16,), _F32)
                    for q4 in range(4):
                        e = g * 4 + q4
                        a = attr_v[b][e, :]
                        for h in range(H):
                            p = (dpack[e, pl.ds(DH * h, 16)]
                                 * kv[b][e, pl.ds(DH * h, 16)]
                                 + dpack[e, pl.ds(DH * h + 16, 16)]
                                 * kv[b][e, pl.ds(DH * h + 16, 16)]
                                 + dpack[e, pl.ds(D + ED * h, 16)] * a)
                            vals = jnp.where(lanes == (q4 * H + h),
                                             jnp.sum(p), vals)
                    ebuf[pl.ds(16 * g, 16)] = vals

                @pl.when(c + 1 < n_chunks)
                def _():
                    pltpu.async_copy(dpack_hbm.at[dst_idx[1 - b]], dpack,
                                     semd)

                @pl.loop(0, CHUNK1 * H // 16)
                def _exp(jj):
                    x = ebuf[pl.ds(jj * 16, 16)]
                    ebuf[pl.ds(jj * 16, 16)] = jnp.exp(x * _INV_SQRT_DH)

                pltpu.async_copy(ebuf.at[pl.ds(0, CHUNK1 * H)],
                                 e_hbm.at[pl.ds(base * H, CHUNK1 * H)], seme)

                @pl.loop(0, CHUNK1)
                def _scale(e):
                    ev = ebuf[pl.ds(H * e, 16)]
                    for h in range(H):
                        s = ev[h]
                        vstage[e, pl.ds(DH * h, 16)] = (
                            kv[b][e, pl.ds(D + DH * h, 16)] * s)
                        vstage[e, pl.ds(DH * h + 16, 16)] = (
                            kv[b][e, pl.ds(D + DH * h + 16, 16)] * s)

                pltpu.sync_copy(vstage, out_sh.at[dst_idx[b]], add=True)

                @pl.when(c + 2 < n_chunks)
                def _():
                    _prefetch(c + 2, b)

        pltpu.make_async_copy(ebuf.at[pl.ds(0, CHUNK1 * H)],
                              e_hbm.at[pl.ds(0, CHUNK1 * H)], seme).wait()
        plsc.subcore_barrier()
        _row_sliced(lambda o, s: pltpu.sync_copy(
            out_sh.at[pl.ds(o, s)], out_hbm.at[cid, pl.ds(o, s)]))

    return sc_pass1


@functools.lru_cache(maxsize=None)
def _sc_pass2_factory(E, n):
    per_worker = E // NW
    n_chunks = per_worker // CHUNK
    assert E == NW * CHUNK * n_chunks and n % 8 == 0 and n_chunks % 2 == 0
    main, rem = _row_partition(n)
    mesh = plsc.VectorSubcoreMesh(core_axis_name="c", subcore_axis_name="s",
                                  num_cores=NC, num_subcores=NS)

    buf2 = lambda *shape_dtype: [pltpu.VMEM(*shape_dtype) for _ in range(2)]

    @functools.partial(
        pl.kernel,
        out_type=jax.ShapeDtypeStruct((NC, n, D), _F32),
        mesh=mesh,
        compiler_params=_SC_PARAMS,
        scratch_types=[
            buf2((CHUNK,), jnp.int32),
            buf2((CHUNK, ED), _F32),
            buf2((CHUNK * H + 16,), _F32),
            pltpu.VMEM((CHUNK, D), _F32),
            pltpu.VMEM_SHARED((n, D), _F32),
            [pltpu.SemaphoreType.DMA for _ in range(2)],
        ],
    )
    def sc_pass2(attr_hbm, dst_hbm, e_hbm, z_nd,
                 aacc_hbm,
                 dst_idx, attr_v, ebuf, stage, acc_sh, sem):
        cid = lax.axis_index("c")
        sid = lax.axis_index("s")
        wid = sid * NC + cid
        r0 = pl.multiple_of(sid * main, 8)
        rt = pl.multiple_of(main * NS + sid * 8, 8)

        def _row_sliced(fn):
            fn(r0, main)
            if rem:
                @pl.when(sid < rem)
                def _():
                    fn(rt, 8)

        _row_sliced(lambda o, s: pltpu.sync_copy(
            z_nd.at[pl.ds(o, s)], acc_sh.at[pl.ds(o, s)]))

        lanes = lax.iota(jnp.int32, 16)
        m4 = lanes < H
        zv = jnp.zeros((16,), _F32)

        @pl.loop(0, CHUNK)
        def _z(e):
            for c in range(QD + 16, D, 16):
                stage[e, pl.ds(c, 16)] = zv

        plsc.subcore_barrier()

        def _copies(c, b):
            base = wid * per_worker + c * CHUNK
            return [(dst_hbm.at[pl.ds(base, CHUNK)], dst_idx[b]),
                    (attr_hbm.at[pl.ds(base, CHUNK)], attr_v[b]),
                    (e_hbm.at[pl.ds(base * H, CHUNK * H)],
                     ebuf[b].at[pl.ds(0, CHUNK * H)])]

        for b in range(2):
            for s_ref, d_ref in _copies(b, b):
                pltpu.async_copy(s_ref, d_ref, sem[b])

        @pl.loop(0, n_chunks // 2)
        def _chunk(j):
            for b in range(2):
                c = 2 * j + b
                for s_ref, d_ref in _copies(c, b):
                    pltpu.make_async_copy(s_ref, d_ref, sem[b]).wait()

                @pl.loop(0, CHUNK)
                def _stage(e):
                    ev = ebuf[b][pl.ds(H * e, 16)]
                    a = attr_v[b][e, :]
                    for h in range(H):
                        stage[e, pl.ds(ED * h, 16)] = a * ev[h]
                    stage[e, pl.ds(QD, 16)] = jnp.where(m4, ev, 0.0)

                pltpu.sync_copy(stage, acc_sh.at[dst_idx[b]], add=True)

                @pl.when(c + 2 < n_chunks)
                def _():
                    for s_ref, d_ref in _copies(c + 2, b):
                        pltpu.async_copy(s_ref, d_ref, sem[b])

        plsc.subcore_barrier()
        _row_sliced(lambda o, s: pltpu.sync_copy(
            acc_sh.at[pl.ds(o, s)], aacc_hbm.at[cid, pl.ds(o, s)]))

    return sc_pass2


def _blockdiag(we, wev):
    wet = jnp.zeros((D, QD), _F32)
    wevt = jnp.zeros((QD, D), _F32)
    for h in range(H):
        wet = wet.at[h * DH:(h + 1) * DH, h * ED:(h + 1) * ED].set(
            we[:, h * DH:(h + 1) * DH].T)
        wevt = wevt.at[h * ED:(h + 1) * ED, h * DH:(h + 1) * DH].set(
            wev[:, h * DH:(h + 1) * DH])
    return wet, wevt


def _att_layer(sc1, sc2, src_x, dst_x, attr, eidx, p, z_nd):
    wet, wevt = _blockdiag(p['We'], p['Wev'])
    dpack, kvpack = _tc_pre(dst_x, src_x, p['Wq'], p['Wk'], p['Wv'], wet)
    out2, e_w = sc1(dpack, kvpack, attr, eidx[0], eidx[1], z_nd)
    aacc2 = sc2(attr, eidx[1], e_w, z_nd)
    r = p['res']
    return _tc_post(out2, aacc2, wevt,
                    r['W0'], r['b0'].reshape(1, D),
                    r['Ws'][0], r['bs'][0].reshape(1, D),
                    r['Ws'][1], r['bs'][1].reshape(1, D))


def kernel(lane_x, movement_x, lane_to_downstream_movement_edge_attr,
           lane_to_upstream_movement_edge_attr, movement_to_movement_edge_attr,
           lane_to_downstream_movement_edge_index,
           lane_to_upstream_movement_edge_index,
           movement_to_movement_edge_index, params):
    n = movement_x.shape[0]
    e = lane_to_downstream_movement_edge_attr.shape[0]
    sc1 = _sc_pass1_factory(e, n)
    sc2 = _sc_pass2_factory(e, n)
    z_nd = jnp.zeros((n, D), _F32)
    inc = _att_layer(sc1, sc2, lane_x, movement_x,
                     lane_to_downstream_movement_edge_attr,
                     lane_to_downstream_movement_edge_index, params['inc'],
                     z_nd)
    outg = _att_layer(sc1, sc2, lane_x, movement_x,
                      lane_to_upstream_movement_edge_attr,
                      lane_to_upstream_movement_edge_index, params['outg'],
                      z_nd)
    c = params['comb']
    mov = _tc_comb(inc, outg, c['W0'][:D], c['W0'][D:],
                   c['b0'].reshape(1, D), c['Ws'][0], c['bs'][0].reshape(1, D),
                   c['Ws'][1], c['bs'][1].reshape(1, D))
    for hp in params['hops']:
        mov = _att_layer(sc1, sc2, mov, mov, movement_to_movement_edge_attr,
                         movement_to_movement_edge_index, hp, z_nd)
    return mov
